# baseline (device time: 22388 ns/iter reference)
import jax
import jax.numpy as jnp
from jax import lax
from jax.experimental import pallas as pl
from jax.experimental.pallas import tpu as pltpu

N_DEV = 4
BLOCK = 64
LOG2E = 1.4426950408889634


def kernel(x, Wq, K_ext, V_ext, Wo):
    B, sq_loc, d_model = x.shape
    _, h_loc = Wq.shape
    _, skv, hq, dh = K_ext.shape
    hq_loc = h_loc // dh

    cd = jnp.bfloat16
    k3 = (K_ext.astype(cd)
          .reshape(B, skv, N_DEV, h_loc)
          .transpose(2, 0, 1, 3))
    vt2 = (V_ext.astype(cd)
           .reshape(B, skv, N_DEV, hq_loc, dh)
           .transpose(2, 0, 3, 4, 1)
           .reshape(N_DEV, B, h_loc, skv))

    def body(x_ref, wq_ref, k2_ref, v_ref, wo_ref, out_ref,
             wq_c, wo_c, qtbd, wq_rx, wo_rx,
             wq_send, wq_recv, wo_send, wo_recv):
        my = lax.axis_index("i")

        wq_c[...] = wq_ref[...].astype(cd)
        wo_c[...] = wo_ref[...].astype(cd)

        barrier = pltpu.get_barrier_semaphore()
        for o in (1, 2, 3):
            pl.semaphore_signal(barrier, inc=1,
                                device_id=(lax.rem(my + o, N_DEV),),
                                device_id_type=pl.DeviceIdType.MESH)
        pl.semaphore_wait(barrier, 3)

        sends = {}
        for o in (1, 3, 2):
            dst = lax.rem(my + o, N_DEV)
            s = 3 - o
            rq = pltpu.make_async_remote_copy(
                src_ref=wq_c, dst_ref=wq_rx.at[s],
                send_sem=wq_send.at[o - 1], recv_sem=wq_recv.at[s],
                device_id=(dst,), device_id_type=pl.DeviceIdType.MESH)
            ro = pltpu.make_async_remote_copy(
                src_ref=wo_c, dst_ref=wo_rx.at[s],
                send_sem=wo_send.at[o - 1], recv_sem=wo_recv.at[s],
                device_id=(dst,), device_id_type=pl.DeviceIdType.MESH)
            rq.start()
            ro.start()
            sends[s] = (rq, ro)

        x_loc = x_ref[...]
        xts = [jnp.transpose(x_loc[b].astype(cd)) for b in range(B)]

        wide = hq_loc * sq_loc
        kvb = lax.broadcasted_iota(jnp.int32, (skv, wide), 0) // BLOCK
        qrow = lax.rem(lax.broadcasted_iota(jnp.int32, (skv, wide), 1),
                       sq_loc) + my * sq_loc
        qb = qrow // BLOCK
        maskw = (qb == kvb) | (kvb == 0) | (lax.rem(qb + kvb, 3) == 0)

        qtbd[...] = jnp.zeros((h_loc, wide), cd)

        def compute_ctx(wq_blk, j):
            ctxts = []
            for b in range(B):
                qt = lax.dot_general(
                    wq_blk, xts[b], (((0,), (0,)), ((), ())),
                    preferred_element_type=jnp.float32)
                qt = (qt * (0.125 * LOG2E)).astype(cd)
                for h in range(hq_loc):
                    qtbd[h * dh:(h + 1) * dh,
                         h * sq_loc:(h + 1) * sq_loc] = \
                        qt[h * dh:(h + 1) * dh, :]
                st = lax.dot(k2_ref[j, b], qtbd[...],
                             preferred_element_type=jnp.float32)
                w = jnp.where(maskw, jnp.exp2(st), 0.0)
                recip = 1.0 / jnp.sum(w, axis=0, keepdims=True)
                pv = lax.dot(v_ref[j, b], w.astype(cd),
                             preferred_element_type=jnp.float32)
                parts = []
                for h in range(hq_loc):
                    c0, c1 = h * sq_loc, (h + 1) * sq_loc
                    parts.append(pv[h * dh:(h + 1) * dh, c0:c1]
                                 * recip[:, c0:c1])
                ctxts.append(jnp.concatenate(parts, axis=0).astype(cd))
            return ctxts

        def apply_out(ctxts, wo_blk, first):
            for b in range(B):
                contrib = lax.dot_general(
                    ctxts[b], wo_blk, (((0,), (0,)), ((), ())),
                    preferred_element_type=jnp.float32)
                if first:
                    out_ref[b, :, :] = contrib
                else:
                    out_ref[b, :, :] = out_ref[b, :, :] + contrib

        apply_out(compute_ctx(wq_c[...], my), wo_c[...], first=True)

        for s in (0, 2, 1):
            rq, ro = sends[s]
            rq.wait_recv()
            ctxts = compute_ctx(wq_rx[s], lax.rem(my + s + 1, N_DEV))
            ro.wait_recv()
            apply_out(ctxts, wo_rx[s], first=False)

        for s in (0, 1, 2):
            rq, ro = sends[s]
            rq.wait_send()
            ro.wait_send()

    return pl.pallas_call(
        body,
        out_shape=jax.ShapeDtypeStruct((B, sq_loc, d_model), jnp.float32),
        in_specs=[pl.BlockSpec(memory_space=pltpu.VMEM)] * 5,
        out_specs=pl.BlockSpec(memory_space=pltpu.VMEM),
        scratch_shapes=[
            pltpu.VMEM((d_model, h_loc), cd),
            pltpu.VMEM((h_loc, d_model), cd),
            pltpu.VMEM((h_loc, hq_loc * sq_loc), cd),
            pltpu.VMEM((N_DEV - 1, d_model, h_loc), cd),
            pltpu.VMEM((N_DEV - 1, h_loc, d_model), cd),
            pltpu.SemaphoreType.DMA((N_DEV - 1,)),
            pltpu.SemaphoreType.DMA((N_DEV - 1,)),
            pltpu.SemaphoreType.DMA((N_DEV - 1,)),
            pltpu.SemaphoreType.DMA((N_DEV - 1,)),
        ],
        compiler_params=pltpu.CompilerParams(collective_id=0),
    )(x, Wq, k3, vt2, Wo)
